# Initial kernel scaffold; baseline (speedup 1.0000x reference)
import jax
import jax.numpy as jnp
from jax.experimental import pallas as pl


def kernel(pred_mask_0, pred_mask_1, pred_mask_2, token_attn_sim_0, token_attn_sim_1, token_attn_sim_2):
    raise NotImplementedError("write your pallas kernel here")


# speedup vs baseline: 4.2203x; 4.2203x over previous
import jax
import jax.numpy as jnp
import numpy as np
from jax.experimental import pallas as pl

_B = 128
_N = 8192
_K = 2457
_ALPHA = 2.0
_NSTAGE = 3
_ROWS = 8
_INT_MIN = np.int32(-(2 ** 31))
_LOW31 = np.int32(0x7FFFFFFF)


def _stage_body(sim_ref, mask_ref, out_ref):
    pid = pl.program_id(0)

    @pl.when(pid == 0)
    def _init():
        out_ref[...] = jnp.zeros((1, 1), jnp.float32)

    s = sim_ref[...]
    bits = jax.lax.bitcast_convert_type(s, jnp.int32)
    key = jnp.where(bits >= 0, bits, bits ^ _LOW31)

    def bit_step(i, p):
        b = 31 - i
        cand = p | (jnp.left_shift(jnp.int32(1), b))
        stest = cand ^ _INT_MIN
        cnt = jnp.sum((key >= stest).astype(jnp.int32), axis=1, keepdims=True)
        return jnp.where(cnt >= _K, cand, p)

    p0 = jnp.zeros((s.shape[0], 1), jnp.int32)
    p = jax.lax.fori_loop(0, 32, bit_step, p0)
    tkey = p ^ _INT_MIN
    tbits = jnp.where(tkey >= 0, tkey, tkey ^ _LOW31)
    t = jax.lax.bitcast_convert_type(tbits, jnp.float32)

    target = (s < t).astype(jnp.float32)
    d = target - mask_ref[...]
    out_ref[...] += jnp.sum(d * d).reshape(1, 1)


def _stage_call(sim, mask):
    return pl.pallas_call(
        _stage_body,
        grid=(_B // _ROWS,),
        in_specs=[
            pl.BlockSpec((_ROWS, _N), lambda i: (i, 0)),
            pl.BlockSpec((_ROWS, _N), lambda i: (i, 0)),
        ],
        out_specs=pl.BlockSpec((1, 1), lambda i: (0, 0)),
        out_shape=jax.ShapeDtypeStruct((1, 1), jnp.float32),
    )(sim, mask)


@jax.jit
def kernel(pred_mask_0, pred_mask_1, pred_mask_2,
           token_attn_sim_0, token_attn_sim_1, token_attn_sim_2):
    total = jnp.float32(0.0)
    for mask, sim3 in ((pred_mask_0, token_attn_sim_0),
                       (pred_mask_1, token_attn_sim_1),
                       (pred_mask_2, token_attn_sim_2)):
        total = total + _stage_call(sim3[:, :, 1], mask)[0, 0]
    return _ALPHA * total / jnp.float32(_NSTAGE * _B * _N)
